# Initial kernel scaffold; baseline (speedup 1.0000x reference)
import jax
import jax.numpy as jnp
from jax.experimental import pallas as pl


def kernel(unary, image):
    raise NotImplementedError("write your pallas kernel here")


# speedup vs baseline: 1.0489x; 1.0489x over previous
import jax
import jax.numpy as jnp
from jax.experimental import pallas as pl
from jax.experimental.pallas import tpu as pltpu

_THETA_ALPHA = 80.0
_THETA_BETA = 13.0
_THETA_GAMMA = 3.0
_BILATERAL_COMPAT = 10.0
_SPATIAL_COMPAT = 3.0
_NUM_ITERATIONS = 5
_BR = 256
_CPAD = 128
_NEG_BIG = 1.0e30


def _build_k_kernel(bi_rows, sp_rows, bi_all, sp_all, sqb_all, sqs_all,
                    kb_ref, ks_ref, nb_ref, ns_ref):
    i = pl.program_id(0)
    n = kb_ref.shape[1]
    rows = i * _BR + jax.lax.broadcasted_iota(jnp.int32, (_BR, n), 0)
    cols = jax.lax.broadcasted_iota(jnp.int32, (_BR, n), 1)
    diag = (rows == cols).astype(jnp.float32)

    def gauss(f_rows, f_all, sq_all):
        fr = f_rows[...]
        sq_r = jnp.sum(fr * fr, axis=1, keepdims=True)
        cross = jax.lax.dot_general(
            fr.astype(jnp.bfloat16), f_all[...].astype(jnp.bfloat16),
            (((1,), (1,)), ((), ())),
            preferred_element_type=jnp.float32)
        d2 = jnp.maximum(sq_r + sq_all[...] - 2.0 * cross, 0.0)
        k16 = (jnp.exp(-0.5 * d2) - diag).astype(jnp.bfloat16)
        norm = jnp.maximum(
            jnp.sum(k16.astype(jnp.float32), axis=1, keepdims=True), 1e-20)
        return k16, norm

    kb_ref[...], nb_ref[...] = gauss(bi_rows, bi_all, sqb_all)
    ks_ref[...], ns_ref[...] = gauss(sp_rows, sp_all, sqs_all)


def _softmax_rows(x):
    m = jnp.max(x, axis=1, keepdims=True)
    e = jnp.exp(x - m)
    return e / jnp.sum(e, axis=1, keepdims=True)


def _mf_kernel(u_ref, kb_ref, ks_ref, nb_ref, ns_ref, out_ref,
               qa_ref, qb_ref):
    it = pl.program_id(0)
    ib = pl.program_id(1)

    @pl.when(jnp.logical_and(it == 0, ib == 0))
    def _init():
        qa_ref[...] = _softmax_rows(-u_ref[...])

    def step(src_ref, dst_ref):
        q16 = src_ref[...].astype(jnp.bfloat16)
        wb = jax.lax.dot_general(
            kb_ref[...], q16, (((1,), (0,)), ((), ())),
            preferred_element_type=jnp.float32)
        ws = jax.lax.dot_general(
            ks_ref[...], q16, (((1,), (0,)), ((), ())),
            preferred_element_type=jnp.float32)
        weighted = (_BILATERAL_COMPAT * (wb / nb_ref[...])
                    + _SPATIAL_COMPAT * (ws / ns_ref[...]))
        q_new = _softmax_rows(weighted - u_ref[pl.ds(ib * _BR, _BR), :])
        dst_ref[pl.ds(ib * _BR, _BR), :] = q_new

        @pl.when(it == _NUM_ITERATIONS - 1)
        def _emit():
            out_ref[pl.ds(ib * _BR, _BR), :] = q_new

    @pl.when(it % 2 == 0)
    def _even():
        step(qa_ref, qb_ref)

    @pl.when(it % 2 == 1)
    def _odd():
        step(qb_ref, qa_ref)


def kernel(unary, image):
    hh, ww, cc = unary.shape
    n = hh * ww
    nblocks = n // _BR

    ys, xs = jnp.meshgrid(jnp.arange(hh, dtype=jnp.float32),
                          jnp.arange(ww, dtype=jnp.float32), indexing='ij')
    xs = xs.reshape(-1)
    ys = ys.reshape(-1)
    rgb = image.reshape(n, 3) * 255.0
    bi = jnp.concatenate([(xs / _THETA_ALPHA)[:, None],
                          (ys / _THETA_ALPHA)[:, None],
                          rgb / _THETA_BETA], axis=1)
    sp = jnp.stack([xs / _THETA_GAMMA, ys / _THETA_GAMMA], axis=1)
    bi = jnp.pad(bi, ((0, 0), (0, 3)))
    sp = jnp.pad(sp, ((0, 0), (0, 6)))
    sqb = jnp.sum(bi * bi, axis=1).reshape(1, n)
    sqs = jnp.sum(sp * sp, axis=1).reshape(1, n)

    kb, ks, nrm_b, nrm_s = pl.pallas_call(
        _build_k_kernel,
        grid=(nblocks,),
        in_specs=[
            pl.BlockSpec((_BR, 8), lambda i: (i, 0)),
            pl.BlockSpec((_BR, 8), lambda i: (i, 0)),
            pl.BlockSpec((n, 8), lambda i: (0, 0)),
            pl.BlockSpec((n, 8), lambda i: (0, 0)),
            pl.BlockSpec((1, n), lambda i: (0, 0)),
            pl.BlockSpec((1, n), lambda i: (0, 0)),
        ],
        out_specs=[
            pl.BlockSpec((_BR, n), lambda i: (i, 0)),
            pl.BlockSpec((_BR, n), lambda i: (i, 0)),
            pl.BlockSpec((_BR, 1), lambda i: (i, 0)),
            pl.BlockSpec((_BR, 1), lambda i: (i, 0)),
        ],
        out_shape=[
            jax.ShapeDtypeStruct((n, n), jnp.bfloat16),
            jax.ShapeDtypeStruct((n, n), jnp.bfloat16),
            jax.ShapeDtypeStruct((n, 1), jnp.float32),
            jax.ShapeDtypeStruct((n, 1), jnp.float32),
        ],
        compiler_params=pltpu.CompilerParams(
            dimension_semantics=("arbitrary",)),
    )(bi, sp, bi, sp, sqb, sqs)

    u = unary.reshape(n, cc)
    u_pad = jnp.full((n, _CPAD), _NEG_BIG, dtype=jnp.float32)
    u_pad = u_pad.at[:, :cc].set(u)

    q = pl.pallas_call(
        _mf_kernel,
        grid=(_NUM_ITERATIONS, nblocks),
        in_specs=[
            pl.BlockSpec((n, _CPAD), lambda it, ib: (0, 0)),
            pl.BlockSpec((_BR, n), lambda it, ib: (ib, 0)),
            pl.BlockSpec((_BR, n), lambda it, ib: (ib, 0)),
            pl.BlockSpec((_BR, 1), lambda it, ib: (ib, 0)),
            pl.BlockSpec((_BR, 1), lambda it, ib: (ib, 0)),
        ],
        out_specs=pl.BlockSpec((n, _CPAD), lambda it, ib: (0, 0)),
        out_shape=jax.ShapeDtypeStruct((n, _CPAD), jnp.float32),
        scratch_shapes=[
            pltpu.VMEM((n, _CPAD), jnp.float32),
            pltpu.VMEM((n, _CPAD), jnp.float32),
        ],
        compiler_params=pltpu.CompilerParams(
            dimension_semantics=("arbitrary", "arbitrary")),
    )(u_pad, kb, ks, nrm_b, nrm_s)

    return q[:, :cc].reshape(hh, ww, cc)


# speedup vs baseline: 1.2150x; 1.1584x over previous
import jax
import jax.numpy as jnp
from jax.experimental import pallas as pl
from jax.experimental.pallas import tpu as pltpu

_THETA_ALPHA = 80.0
_THETA_BETA = 13.0
_THETA_GAMMA = 3.0
_BILATERAL_COMPAT = 10.0
_SPATIAL_COMPAT = 3.0
_NUM_ITERATIONS = 5
_BR = 256
_CPAD = 128
_ONES_LANE = 21
_NEG_BIG = 1.0e30


def _softmax_rows(x):
    m = jnp.max(x, axis=1, keepdims=True)
    e = jnp.exp(x - m)
    return e / jnp.sum(e, axis=1, keepdims=True)


def _gauss_strip(row0, f16_ref, sq_row_ref, sq_col_ref, n):
    rows = row0 + jax.lax.broadcasted_iota(jnp.int32, (_BR, n), 0)
    cols = jax.lax.broadcasted_iota(jnp.int32, (_BR, n), 1)
    diag = (rows == cols).astype(jnp.float32)
    cross = jax.lax.dot_general(
        f16_ref[pl.ds(row0, _BR), :], f16_ref[...],
        (((1,), (1,)), ((), ())),
        preferred_element_type=jnp.float32)
    d2 = jnp.maximum(
        sq_row_ref[pl.ds(row0, _BR), :] + sq_col_ref[...] - 2.0 * cross, 0.0)
    return (jnp.exp(-0.5 * d2) - diag).astype(jnp.bfloat16)


def _build_ksp_kernel(sp16_ref, sqs_row_ref, sqs_col_ref, ksp_ref):
    ksp_ref[...] = _gauss_strip(pl.program_id(0) * _BR, sp16_ref,
                                sqs_row_ref, sqs_col_ref, ksp_ref.shape[1])


def _crf_kernel(bi16_ref, sqb_row_ref, sqb_col_ref, u_ref, ksp_ref, out_ref,
                kb_ref, qa_ref, qb_ref):
    it = pl.program_id(0)
    ib = pl.program_id(1)
    n = kb_ref.shape[1]
    row0 = ib * _BR
    lane = jax.lax.broadcasted_iota(jnp.int32, (_BR, _CPAD), 1)

    @pl.when(it == 0)
    def _build():
        kb_ref[pl.ds(row0, _BR), :] = _gauss_strip(
            row0, bi16_ref, sqb_row_ref, sqb_col_ref, n)

        @pl.when(ib == 0)
        def _init():
            q0 = _softmax_rows(-u_ref[...])
            ln = jax.lax.broadcasted_iota(jnp.int32, q0.shape, 1)
            qa_ref[...] = jnp.where(ln == _ONES_LANE, 1.0, q0
                                    ).astype(jnp.bfloat16)

    @pl.when(it > 0)
    def _iterate():
        def step(src_ref, dst_ref):
            q16 = src_ref[...]
            wb = jax.lax.dot_general(
                kb_ref[pl.ds(row0, _BR), :], q16, (((1,), (0,)), ((), ())),
                preferred_element_type=jnp.float32)
            ws = jax.lax.dot_general(
                ksp_ref[...], q16, (((1,), (0,)), ((), ())),
                preferred_element_type=jnp.float32)
            nb = jnp.maximum(wb[:, _ONES_LANE:_ONES_LANE + 1], 1e-20)
            ns = jnp.maximum(ws[:, _ONES_LANE:_ONES_LANE + 1], 1e-20)
            weighted = (_BILATERAL_COMPAT * (wb / nb)
                        + _SPATIAL_COMPAT * (ws / ns))
            q_new = _softmax_rows(weighted - u_ref[pl.ds(row0, _BR), :])
            dst_ref[pl.ds(row0, _BR), :] = jnp.where(
                lane == _ONES_LANE, 1.0, q_new).astype(jnp.bfloat16)

            @pl.when(it == _NUM_ITERATIONS)
            def _emit():
                out_ref[pl.ds(row0, _BR), :] = q_new

        @pl.when(it % 2 == 1)
        def _odd():
            step(qa_ref, qb_ref)

        @pl.when(jnp.logical_and(it % 2 == 0, it > 0))
        def _even():
            step(qb_ref, qa_ref)


def kernel(unary, image):
    hh, ww, cc = unary.shape
    n = hh * ww
    nblocks = n // _BR

    ys, xs = jnp.meshgrid(jnp.arange(hh, dtype=jnp.float32),
                          jnp.arange(ww, dtype=jnp.float32), indexing='ij')
    xs = xs.reshape(-1)
    ys = ys.reshape(-1)
    rgb = image.reshape(n, 3) * 255.0
    bi = jnp.concatenate([(xs / _THETA_ALPHA)[:, None],
                          (ys / _THETA_ALPHA)[:, None],
                          rgb / _THETA_BETA], axis=1)
    sp = jnp.stack([xs / _THETA_GAMMA, ys / _THETA_GAMMA], axis=1)
    bi = jnp.pad(bi, ((0, 0), (0, 3)))
    sp = jnp.pad(sp, ((0, 0), (0, 6)))
    sqb = jnp.sum(bi * bi, axis=1)
    sqs = jnp.sum(sp * sp, axis=1)

    ksp = pl.pallas_call(
        _build_ksp_kernel,
        grid=(nblocks,),
        in_specs=[
            pl.BlockSpec((n, 8), lambda i: (0, 0)),
            pl.BlockSpec((n, 1), lambda i: (0, 0)),
            pl.BlockSpec((1, n), lambda i: (0, 0)),
        ],
        out_specs=pl.BlockSpec((_BR, n), lambda i: (i, 0)),
        out_shape=jax.ShapeDtypeStruct((n, n), jnp.bfloat16),
        compiler_params=pltpu.CompilerParams(
            dimension_semantics=("arbitrary",)),
    )(sp.astype(jnp.bfloat16), sqs.reshape(n, 1), sqs.reshape(1, n))

    u = unary.reshape(n, cc)
    u_pad = jnp.full((n, _CPAD), _NEG_BIG, dtype=jnp.float32)
    u_pad = u_pad.at[:, :cc].set(u)

    q = pl.pallas_call(
        _crf_kernel,
        grid=(1 + _NUM_ITERATIONS, nblocks),
        in_specs=[
            pl.BlockSpec((n, 8), lambda it, ib: (0, 0)),
            pl.BlockSpec((n, 1), lambda it, ib: (0, 0)),
            pl.BlockSpec((1, n), lambda it, ib: (0, 0)),
            pl.BlockSpec((n, _CPAD), lambda it, ib: (0, 0)),
            pl.BlockSpec((_BR, n), lambda it, ib: (ib, 0)),
        ],
        out_specs=pl.BlockSpec((n, _CPAD), lambda it, ib: (0, 0)),
        out_shape=jax.ShapeDtypeStruct((n, _CPAD), jnp.float32),
        scratch_shapes=[
            pltpu.VMEM((n, n), jnp.bfloat16),
            pltpu.VMEM((n, _CPAD), jnp.bfloat16),
            pltpu.VMEM((n, _CPAD), jnp.bfloat16),
        ],
        compiler_params=pltpu.CompilerParams(
            dimension_semantics=("arbitrary", "arbitrary")),
    )(bi.astype(jnp.bfloat16), sqb.reshape(n, 1), sqb.reshape(1, n),
      u_pad, ksp)

    return q[:, :cc].reshape(hh, ww, cc)


# speedup vs baseline: 1.4832x; 1.2208x over previous
import jax
import jax.numpy as jnp
from jax.experimental import pallas as pl
from jax.experimental.pallas import tpu as pltpu

_THETA_ALPHA = 80.0
_THETA_BETA = 13.0
_THETA_GAMMA = 3.0
_BILATERAL_COMPAT = 10.0
_SPATIAL_COMPAT = 3.0
_NUM_ITERATIONS = 5
_BR = 256
_CPAD = 128
_NEG_BIG = 1.0e30


def _softmax_rows(x):
    m = jnp.max(x, axis=1, keepdims=True)
    e = jnp.exp(x - m)
    return e / jnp.sum(e, axis=1, keepdims=True)


def _crf_kernel(bi16_ref, sp16_ref, sqb_row_ref, sqb_col_ref, sqs_row_ref,
                sqs_col_ref, u_ref, out_ref, m_ref, qa_ref, qb_ref):
    it = pl.program_id(0)
    ib = pl.program_id(1)
    n = m_ref.shape[1]
    row0 = ib * _BR

    @pl.when(it == 0)
    def _build():
        rows = row0 + jax.lax.broadcasted_iota(jnp.int32, (_BR, n), 0)
        cols = jax.lax.broadcasted_iota(jnp.int32, (_BR, n), 1)
        diag = (rows == cols).astype(jnp.float32)

        def gauss(f16_ref, sq_row_ref, sq_col_ref):
            cross = jax.lax.dot_general(
                f16_ref[pl.ds(row0, _BR), :], f16_ref[...],
                (((1,), (1,)), ((), ())),
                preferred_element_type=jnp.float32)
            d2 = jnp.maximum(
                sq_row_ref[pl.ds(row0, _BR), :] + sq_col_ref[...]
                - 2.0 * cross, 0.0)
            k = (jnp.exp(-0.5 * d2) - diag).astype(jnp.bfloat16
                                                   ).astype(jnp.float32)
            norm = jnp.maximum(jnp.sum(k, axis=1, keepdims=True), 1e-20)
            return k, norm

        kb, nb = gauss(bi16_ref, sqb_row_ref, sqb_col_ref)
        ks, ns = gauss(sp16_ref, sqs_row_ref, sqs_col_ref)
        m_ref[pl.ds(row0, _BR), :] = (
            (_BILATERAL_COMPAT / nb) * kb + (_SPATIAL_COMPAT / ns) * ks
        ).astype(jnp.bfloat16)

        @pl.when(ib == 0)
        def _init():
            qa_ref[...] = _softmax_rows(-u_ref[...]).astype(jnp.bfloat16)

    @pl.when(it > 0)
    def _iterate():
        def step(src_ref, dst_ref):
            wm = jax.lax.dot_general(
                m_ref[pl.ds(row0, _BR), :], src_ref[...],
                (((1,), (0,)), ((), ())),
                preferred_element_type=jnp.float32)
            q_new = _softmax_rows(wm - u_ref[pl.ds(row0, _BR), :])
            dst_ref[pl.ds(row0, _BR), :] = q_new.astype(jnp.bfloat16)

            @pl.when(it == _NUM_ITERATIONS)
            def _emit():
                out_ref[pl.ds(row0, _BR), :] = q_new

        @pl.when(it % 2 == 1)
        def _odd():
            step(qa_ref, qb_ref)

        @pl.when(jnp.logical_and(it % 2 == 0, it > 0))
        def _even():
            step(qb_ref, qa_ref)


def kernel(unary, image):
    hh, ww, cc = unary.shape
    n = hh * ww
    nblocks = n // _BR

    ys, xs = jnp.meshgrid(jnp.arange(hh, dtype=jnp.float32),
                          jnp.arange(ww, dtype=jnp.float32), indexing='ij')
    xs = xs.reshape(-1)
    ys = ys.reshape(-1)
    rgb = image.reshape(n, 3) * 255.0
    bi = jnp.concatenate([(xs / _THETA_ALPHA)[:, None],
                          (ys / _THETA_ALPHA)[:, None],
                          rgb / _THETA_BETA], axis=1)
    sp = jnp.stack([xs / _THETA_GAMMA, ys / _THETA_GAMMA], axis=1)
    bi = jnp.pad(bi, ((0, 0), (0, 3)))
    sp = jnp.pad(sp, ((0, 0), (0, 6)))
    sqb = jnp.sum(bi * bi, axis=1)
    sqs = jnp.sum(sp * sp, axis=1)

    u = unary.reshape(n, cc)
    u_pad = jnp.full((n, _CPAD), _NEG_BIG, dtype=jnp.float32)
    u_pad = u_pad.at[:, :cc].set(u)

    q = pl.pallas_call(
        _crf_kernel,
        grid=(1 + _NUM_ITERATIONS, nblocks),
        in_specs=[
            pl.BlockSpec((n, 8), lambda it, ib: (0, 0)),
            pl.BlockSpec((n, 8), lambda it, ib: (0, 0)),
            pl.BlockSpec((n, 1), lambda it, ib: (0, 0)),
            pl.BlockSpec((1, n), lambda it, ib: (0, 0)),
            pl.BlockSpec((n, 1), lambda it, ib: (0, 0)),
            pl.BlockSpec((1, n), lambda it, ib: (0, 0)),
            pl.BlockSpec((n, _CPAD), lambda it, ib: (0, 0)),
        ],
        out_specs=pl.BlockSpec((n, _CPAD), lambda it, ib: (0, 0)),
        out_shape=jax.ShapeDtypeStruct((n, _CPAD), jnp.float32),
        scratch_shapes=[
            pltpu.VMEM((n, n), jnp.bfloat16),
            pltpu.VMEM((n, _CPAD), jnp.bfloat16),
            pltpu.VMEM((n, _CPAD), jnp.bfloat16),
        ],
        compiler_params=pltpu.CompilerParams(
            dimension_semantics=("arbitrary", "arbitrary")),
    )(bi.astype(jnp.bfloat16), sp.astype(jnp.bfloat16),
      sqb.reshape(n, 1), sqb.reshape(1, n),
      sqs.reshape(n, 1), sqs.reshape(1, n), u_pad)

    return q[:, :cc].reshape(hh, ww, cc)


# speedup vs baseline: 2.0083x; 1.3540x over previous
import jax
import jax.numpy as jnp
from jax.experimental import pallas as pl
from jax.experimental.pallas import tpu as pltpu

_THETA_ALPHA = 80.0
_THETA_BETA = 13.0
_THETA_GAMMA = 3.0
_BILATERAL_COMPAT = 10.0
_SPATIAL_COMPAT = 3.0
_NUM_ITERATIONS = 5
_BR = 256
_CPAD = 128
_NEG_BIG = 1.0e30


def _softmax_rows(x):
    m = jnp.max(x, axis=1, keepdims=True)
    e = jnp.exp(x - m)
    return e / jnp.sum(e, axis=1, keepdims=True)


def _crf_kernel(bi16_ref, sp16_ref, nsqb_row_ref, nsqb_col_ref, nsqs_row_ref,
                nsqs_col_ref, u_ref, out_ref, m_ref, qa_ref, qb_ref):
    p = pl.program_id(0)
    n = m_ref.shape[1]
    nblocks = n // _BR

    @pl.when(p < nblocks)
    def _build():
        row0 = p * _BR
        rows = row0 + jax.lax.broadcasted_iota(jnp.int32, (_BR, n), 0)
        cols = jax.lax.broadcasted_iota(jnp.int32, (_BR, n), 1)
        diag = (rows == cols).astype(jnp.float32)

        def gauss(f16_ref, nsq_row_ref, nsq_col_ref):
            cross = jax.lax.dot_general(
                f16_ref[pl.ds(row0, _BR), :], f16_ref[...],
                (((1,), (1,)), ((), ())),
                preferred_element_type=jnp.float32)
            karg = jnp.minimum(
                nsq_row_ref[pl.ds(row0, _BR), :] + nsq_col_ref[...] + cross,
                0.0)
            k = jnp.exp(karg) - diag
            norm = jnp.maximum(jnp.sum(k, axis=1, keepdims=True), 1e-20)
            return k, norm

        kb, nb = gauss(bi16_ref, nsqb_row_ref, nsqb_col_ref)
        ks, ns = gauss(sp16_ref, nsqs_row_ref, nsqs_col_ref)
        m_ref[pl.ds(row0, _BR), :] = (
            (_BILATERAL_COMPAT / nb) * kb + (_SPATIAL_COMPAT / ns) * ks
        ).astype(jnp.bfloat16)

        @pl.when(p == 0)
        def _init():
            qa_ref[...] = _softmax_rows(-u_ref[...]).astype(jnp.bfloat16)

    @pl.when(p >= nblocks)
    def _iterate():
        it = p - nblocks

        def step(src_ref, dst_ref):
            wm = jax.lax.dot_general(
                m_ref[...], src_ref[...], (((1,), (0,)), ((), ())),
                preferred_element_type=jnp.float32)
            q_new = _softmax_rows(wm - u_ref[...])
            dst_ref[...] = q_new.astype(jnp.bfloat16)

            @pl.when(it == _NUM_ITERATIONS - 1)
            def _emit():
                out_ref[...] = q_new

        @pl.when(it % 2 == 0)
        def _even():
            step(qa_ref, qb_ref)

        @pl.when(it % 2 == 1)
        def _odd():
            step(qb_ref, qa_ref)


def kernel(unary, image):
    hh, ww, cc = unary.shape
    n = hh * ww
    nblocks = n // _BR

    ys, xs = jnp.meshgrid(jnp.arange(hh, dtype=jnp.float32),
                          jnp.arange(ww, dtype=jnp.float32), indexing='ij')
    xs = xs.reshape(-1)
    ys = ys.reshape(-1)
    rgb = image.reshape(n, 3) * 255.0
    bi = jnp.concatenate([(xs / _THETA_ALPHA)[:, None],
                          (ys / _THETA_ALPHA)[:, None],
                          rgb / _THETA_BETA], axis=1)
    sp = jnp.stack([xs / _THETA_GAMMA, ys / _THETA_GAMMA], axis=1)
    bi = jnp.pad(bi, ((0, 0), (0, 3)))
    sp = jnp.pad(sp, ((0, 0), (0, 6)))
    nsqb = -0.5 * jnp.sum(bi * bi, axis=1)
    nsqs = -0.5 * jnp.sum(sp * sp, axis=1)

    u = unary.reshape(n, cc)
    u_pad = jnp.full((n, _CPAD), _NEG_BIG, dtype=jnp.float32)
    u_pad = u_pad.at[:, :cc].set(u)

    q = pl.pallas_call(
        _crf_kernel,
        grid=(nblocks + _NUM_ITERATIONS,),
        in_specs=[
            pl.BlockSpec((n, 8), lambda p: (0, 0)),
            pl.BlockSpec((n, 8), lambda p: (0, 0)),
            pl.BlockSpec((n, 1), lambda p: (0, 0)),
            pl.BlockSpec((1, n), lambda p: (0, 0)),
            pl.BlockSpec((n, 1), lambda p: (0, 0)),
            pl.BlockSpec((1, n), lambda p: (0, 0)),
            pl.BlockSpec((n, _CPAD), lambda p: (0, 0)),
        ],
        out_specs=pl.BlockSpec((n, _CPAD), lambda p: (0, 0)),
        out_shape=jax.ShapeDtypeStruct((n, _CPAD), jnp.float32),
        scratch_shapes=[
            pltpu.VMEM((n, n), jnp.bfloat16),
            pltpu.VMEM((n, _CPAD), jnp.bfloat16),
            pltpu.VMEM((n, _CPAD), jnp.bfloat16),
        ],
        compiler_params=pltpu.CompilerParams(
            dimension_semantics=("arbitrary",)),
    )(bi.astype(jnp.bfloat16), sp.astype(jnp.bfloat16),
      nsqb.reshape(n, 1), nsqb.reshape(1, n),
      nsqs.reshape(n, 1), nsqs.reshape(1, n), u_pad)

    return q[:, :cc].reshape(hh, ww, cc)


# speedup vs baseline: 2.2533x; 1.1220x over previous
import jax
import jax.numpy as jnp
from jax.experimental import pallas as pl
from jax.experimental.pallas import tpu as pltpu

_THETA_ALPHA = 80.0
_THETA_BETA = 13.0
_THETA_GAMMA = 3.0
_BILATERAL_COMPAT = 10.0
_SPATIAL_COMPAT = 3.0
_NUM_ITERATIONS = 5
_BR = 256
_CPAD = 128
_NEG_BIG = 1.0e30


def _softmax_rows(x):
    m = jnp.max(x, axis=1, keepdims=True)
    e = jnp.exp(x - m)
    return e / jnp.sum(e, axis=1, keepdims=True)


def _crf_kernel(bi16_ref, sp16_ref, nsqb_col_ref, nsqs_col_ref,
                u_ref, out_ref, m_ref, c_ref, qa_ref, qb_ref):
    p = pl.program_id(0)
    n = m_ref.shape[1]
    nblocks = n // _BR

    @pl.when(p < nblocks)
    def _build():
        row0 = p * _BR

        def gauss(f16_ref, nsq_col_ref):
            cross = jax.lax.dot_general(
                f16_ref[pl.ds(row0, _BR), :], f16_ref[...],
                (((1,), (1,)), ((), ())),
                preferred_element_type=jnp.float32)
            nsq_row = jnp.reshape(nsq_col_ref[:, pl.ds(row0, _BR)], (_BR, 1))
            e = jnp.exp(jnp.minimum(nsq_row + nsq_col_ref[...] + cross, 0.0))
            norm = jnp.maximum(
                jnp.sum(e, axis=1, keepdims=True) - 1.0, 1e-20)
            return e, norm

        eb, nb = gauss(bi16_ref, nsqb_col_ref)
        es, ns = gauss(sp16_ref, nsqs_col_ref)
        sb = _BILATERAL_COMPAT / nb
        ss = _SPATIAL_COMPAT / ns
        m_ref[pl.ds(row0, _BR), :] = (sb * eb + ss * es).astype(jnp.bfloat16)
        c_ref[pl.ds(row0, _BR), :] = (sb + ss).astype(jnp.bfloat16)

        @pl.when(p == 0)
        def _init():
            qa_ref[...] = _softmax_rows(-u_ref[...]).astype(jnp.bfloat16)

    @pl.when(p >= nblocks)
    def _iterate():
        it = p - nblocks

        def step(src_ref, dst_ref):
            q16 = src_ref[...]
            wm = jax.lax.dot_general(
                m_ref[...], q16, (((1,), (0,)), ((), ())),
                preferred_element_type=jnp.float32)
            weighted = wm - c_ref[...].astype(jnp.float32) * q16.astype(jnp.float32)
            q_new = _softmax_rows(weighted - u_ref[...])
            dst_ref[...] = q_new.astype(jnp.bfloat16)

            @pl.when(it == _NUM_ITERATIONS - 1)
            def _emit():
                out_ref[...] = q_new

        @pl.when(it % 2 == 0)
        def _even():
            step(qa_ref, qb_ref)

        @pl.when(it % 2 == 1)
        def _odd():
            step(qb_ref, qa_ref)


def kernel(unary, image):
    hh, ww, cc = unary.shape
    n = hh * ww
    nblocks = n // _BR

    ys, xs = jnp.meshgrid(jnp.arange(hh, dtype=jnp.float32),
                          jnp.arange(ww, dtype=jnp.float32), indexing='ij')
    xs = xs.reshape(-1)
    ys = ys.reshape(-1)
    rgb = image.reshape(n, 3) * 255.0
    bi = jnp.concatenate([(xs / _THETA_ALPHA)[:, None],
                          (ys / _THETA_ALPHA)[:, None],
                          rgb / _THETA_BETA], axis=1)
    sp = jnp.stack([xs / _THETA_GAMMA, ys / _THETA_GAMMA], axis=1)
    bi = jnp.pad(bi, ((0, 0), (0, 3)))
    sp = jnp.pad(sp, ((0, 0), (0, 6)))
    nsqb = -0.5 * jnp.sum(bi * bi, axis=1)
    nsqs = -0.5 * jnp.sum(sp * sp, axis=1)

    u = unary.reshape(n, cc)
    u_pad = jnp.full((n, _CPAD), _NEG_BIG, dtype=jnp.float32)
    u_pad = u_pad.at[:, :cc].set(u)

    q = pl.pallas_call(
        _crf_kernel,
        grid=(nblocks + _NUM_ITERATIONS,),
        in_specs=[
            pl.BlockSpec((n, 8), lambda p: (0, 0)),
            pl.BlockSpec((n, 8), lambda p: (0, 0)),
            pl.BlockSpec((1, n), lambda p: (0, 0)),
            pl.BlockSpec((1, n), lambda p: (0, 0)),
            pl.BlockSpec((n, _CPAD), lambda p: (0, 0)),
        ],
        out_specs=pl.BlockSpec((n, _CPAD), lambda p: (0, 0)),
        out_shape=jax.ShapeDtypeStruct((n, _CPAD), jnp.float32),
        scratch_shapes=[
            pltpu.VMEM((n, n), jnp.bfloat16),
            pltpu.VMEM((n, 1), jnp.bfloat16),
            pltpu.VMEM((n, _CPAD), jnp.bfloat16),
            pltpu.VMEM((n, _CPAD), jnp.bfloat16),
        ],
        compiler_params=pltpu.CompilerParams(
            dimension_semantics=("arbitrary",)),
    )(bi.astype(jnp.bfloat16), sp.astype(jnp.bfloat16),
      nsqb.reshape(1, n), nsqs.reshape(1, n), u_pad)

    return q[:, :cc].reshape(hh, ww, cc)
